# baseline (speedup 1.0000x reference)
import jax
import jax.numpy as jnp
from jax import lax
from jax.experimental import pallas as pl
from jax.experimental.pallas import tpu as pltpu
from jax.experimental.pallas import tpu_sc as plsc

N = 10000
E = 160000
D_IN = 256
HID = 512
LANES = 128
NC = 2
NS = 16
K = 80
B = 128
E_PAD = NS * B * K
NGRP = B // 4
NP_ = 10112
RPT = NP_ // NS
BN = 2000


def _make_sc_agg(C, with_cnt):
    out_type = [jax.ShapeDtypeStruct((C * NP_, LANES), jnp.float32)]
    if with_cnt:
        out_type.append(jax.ShapeDtypeStruct((NC, NP_), jnp.float32))
    mesh = plsc.VectorSubcoreMesh(core_axis_name="c", subcore_axis_name="s",
                                  num_cores=NC, num_subcores=NS)
    scratch_types = [
        *[pltpu.VMEM((2, 4, K), jnp.int32) for _ in range(2)],
        *[pltpu.VMEM((K, LANES), jnp.float32) for _ in range(2)],
        pltpu.VMEM((K,), jnp.float32),
        pltpu.VMEM_SHARED((NP_, LANES), jnp.float32),
        pltpu.VMEM_SHARED((NP_,), jnp.float32),
        *[pltpu.SemaphoreType.DMA for _ in range(6)],
    ]

    def body(xs, egrp, z2d, z1d, onesk, *rest):
        if with_cnt:
            agg_o, cnt_o = rest[0], rest[1]
            rest = rest[2:]
        else:
            agg_o = rest[0]
            rest = rest[1:]
        egb = rest[0:2]
        rows = rest[2:4]
        onev, agg_sp, cnt_sp = rest[4:7]
        isems = rest[7:9]
        gsems = rest[9:11]
        ssems = rest[11:13]
        ci = lax.axis_index("c")
        si = lax.axis_index("s")
        r0 = si * RPT

        if with_cnt:
            pltpu.sync_copy(onesk, onev)

        def cnt_on(i):
            return jnp.logical_xor(ci == 1, i < B // 2)

        for cc in range(C // NC):
            chunk = cc * NC + ci
            xs_c = xs.at[pl.ds(pl.multiple_of(chunk * N, 8), N)]

            def grp_issue(q, j):
                pltpu.async_copy(egrp.at[si, j], egb[q], isems[q])

            def grp_wait(q, j):
                pltpu.make_async_copy(egrp.at[si, j], egb[q], isems[q]).wait()

            def g_issue(q, p4, b):
                pltpu.async_copy(xs_c.at[egb[q].at[0, p4]], rows[b], gsems[b])

            def g_wait(q, p4, b):
                pltpu.make_async_copy(xs_c.at[egb[q].at[0, p4]], rows[b],
                                      gsems[b]).wait()

            def s_issue(q, p4, b, i):
                pltpu.async_copy(rows[b], agg_sp.at[egb[q].at[1, p4]],
                                 ssems[b], add=True)
                if with_cnt:
                    @pl.when(cnt_on(i))
                    def _():
                        pltpu.async_copy(onev, cnt_sp.at[egb[q].at[1, p4]],
                                         ssems[b], add=True)

            def s_wait(q, p4, b, i):
                pltpu.make_async_copy(rows[b], agg_sp.at[egb[q].at[1, p4]],
                                      ssems[b]).wait()
                if with_cnt:
                    @pl.when(cnt_on(i))
                    def _():
                        pltpu.make_async_copy(onev, cnt_sp.at[egb[q].at[1, p4]],
                                              ssems[b]).wait()

            pltpu.sync_copy(z2d.at[pl.ds(r0, RPT)], agg_sp.at[pl.ds(r0, RPT)])
            if with_cnt:
                @pl.when(si == 0)
                def _():
                    pltpu.sync_copy(z1d, cnt_sp)
            plsc.subcore_barrier()

            grp_issue(0, 0)

            def step(g, _):
                for q in range(2):
                    j = 2 * g + q
                    grp_wait(q, j)
                    for p4 in range(4):
                        i = 4 * j + p4
                        b = p4 % 2

                        @pl.when(i >= 2)
                        def _():
                            qq2 = q if p4 >= 2 else 1 - q
                            s_wait(qq2, (p4 - 2) % 4, b, i - 2)
                        g_issue(q, p4, b)
                        qq, pp = (q, p4 - 1) if p4 else (1 - q, 3)

                        @pl.when(i >= 1)
                        def _():
                            g_wait(qq, pp, 1 - b)
                            s_issue(qq, pp, 1 - b, i - 1)

                    @pl.when(j + 1 < NGRP)
                    def _():
                        grp_issue(1 - q, j + 1)
                return 0

            lax.fori_loop(0, NGRP // 2, step, 0)
            g_wait(1, 3, 1)
            s_issue(1, 3, 1, B - 1)
            s_wait(1, 2, 0, B - 2)
            s_wait(1, 3, 1, B - 1)
            plsc.subcore_barrier()

            o0 = chunk * NP_ + r0
            pltpu.sync_copy(agg_sp.at[pl.ds(r0, RPT)], agg_o.at[pl.ds(o0, RPT)])
            if with_cnt:
                @pl.when(si == 0)
                def _():
                    pltpu.sync_copy(cnt_sp, cnt_o.at[ci])
            if cc + 1 < C // NC:
                plsc.subcore_barrier()

    return pl.kernel(body, out_type=out_type, mesh=mesh,
                     scratch_types=scratch_types)


_sc_agg2 = _make_sc_agg(2, with_cnt=True)
_sc_agg4 = _make_sc_agg(4, with_cnt=False)


def _tc0_body(agg_ref, cnt_ref, x_ref, wl_ref, wr_ref, b_ref, out_ref):
    inv = 1.0 / jnp.maximum(cnt_ref[0] + cnt_ref[1], 1.0)
    wl = wl_ref[...]
    acc = jnp.dot(x_ref[...], wr_ref[...], preferred_element_type=jnp.float32)
    for c in range(D_IN // LANES):
        mean_c = (agg_ref[c] * inv).astype(jnp.bfloat16)
        acc += jnp.dot(mean_c, wl[c * LANES:(c + 1) * LANES, :],
                       preferred_element_type=jnp.float32)
    h = jnp.maximum(acc + b_ref[...], 0.0)
    for c in range(HID // LANES):
        out_ref[c] = h[:, c * LANES:(c + 1) * LANES]


_tc_layer0 = pl.pallas_call(
    _tc0_body,
    grid=(N // BN,),
    in_specs=[
        pl.BlockSpec((D_IN // LANES, BN, LANES), lambda i: (0, i, 0)),
        pl.BlockSpec((NC, BN, 1), lambda i: (0, i, 0)),
        pl.BlockSpec((BN, D_IN), lambda i: (i, 0)),
        pl.BlockSpec((D_IN, HID), lambda i: (0, 0)),
        pl.BlockSpec((D_IN, HID), lambda i: (0, 0)),
        pl.BlockSpec((1, HID), lambda i: (0, 0)),
    ],
    out_specs=pl.BlockSpec((HID // LANES, BN, LANES), lambda i: (0, i, 0)),
    out_shape=jax.ShapeDtypeStruct((HID // LANES, N, LANES), jnp.float32),
)


def _tc1_body(agg_ref, cnt_ref, h_ref, wl_ref, wr_ref, b_ref, out_ref):
    inv = 1.0 / jnp.maximum(cnt_ref[0] + cnt_ref[1], 1.0)
    wl = wl_ref[...]
    wr = wr_ref[...]
    acc = jnp.broadcast_to(b_ref[...], (BN, HID))
    for c in range(HID // LANES):
        mean_c = (agg_ref[c] * inv).astype(jnp.bfloat16)
        acc += jnp.dot(mean_c, wl[c * LANES:(c + 1) * LANES, :],
                       preferred_element_type=jnp.float32)
        acc += jnp.dot(h_ref[c].astype(jnp.bfloat16),
                       wr[c * LANES:(c + 1) * LANES, :],
                       preferred_element_type=jnp.float32)
    out_ref[...] = jnp.maximum(acc, 0.0)


_tc_layer1 = pl.pallas_call(
    _tc1_body,
    grid=(N // BN,),
    in_specs=[
        pl.BlockSpec((HID // LANES, BN, LANES), lambda i: (0, i, 0)),
        pl.BlockSpec((NC, BN, 1), lambda i: (0, i, 0)),
        pl.BlockSpec((HID // LANES, BN, LANES), lambda i: (0, i, 0)),
        pl.BlockSpec((HID, HID), lambda i: (0, 0)),
        pl.BlockSpec((HID, HID), lambda i: (0, 0)),
        pl.BlockSpec((1, HID), lambda i: (0, 0)),
    ],
    out_specs=pl.BlockSpec((BN, HID), lambda i: (i, 0)),
    out_shape=jax.ShapeDtypeStruct((N, HID), jnp.float32),
)


def kernel(x, edge_index, W0_l, W0_r, b0, W1_l, W1_r, b1):
    xs0 = x.reshape(N, D_IN // LANES, LANES).transpose(1, 0, 2)
    xs0 = xs0.reshape((D_IN // LANES) * N, LANES)
    z2d = jnp.zeros((NP_, LANES), jnp.float32)
    z1d = jnp.zeros((NP_,), jnp.float32)
    onesk = jnp.ones((K,), jnp.float32)

    pad_src = jnp.zeros((E_PAD - E,), jnp.int32)
    pad_dst = jnp.full((E_PAD - E,), NP_ - 1, jnp.int32)
    ei = jnp.concatenate([edge_index,
                          jnp.stack([pad_src, pad_dst])], axis=1)
    egrp = ei.reshape(2, NS, NGRP, 4, K).transpose(1, 2, 0, 3, 4)
    agg0, cnt = _sc_agg2(xs0, egrp, z2d, z1d, onesk)
    cnt2 = cnt.reshape(NC, NP_, 1)
    bf = jnp.bfloat16
    h_split = _tc_layer0(agg0.reshape(D_IN // LANES, NP_, LANES), cnt2,
                         x.astype(bf), W0_l.astype(bf), W0_r.astype(bf),
                         b0.reshape(1, HID))
    agg1, = _sc_agg4(h_split.reshape((HID // LANES) * N, LANES), egrp,
                     z2d, z1d, onesk)
    out = _tc_layer1(agg1.reshape(HID // LANES, NP_, LANES), cnt2, h_split,
                     W1_l.astype(bf), W1_r.astype(bf), b1.reshape(1, HID))
    return out


# speedup vs baseline: 2.1497x; 2.1497x over previous
import jax
import jax.numpy as jnp
from jax import lax
from jax.experimental import pallas as pl
from jax.experimental.pallas import tpu as pltpu
from jax.experimental.pallas import tpu_sc as plsc

N = 10000
E = 160000
D_IN = 256
HID = 512
LANES = 128
NC = 2
NS = 16
K = 80
B = 125
G = 5
NGRP = B // G
NP_ = 10112
RPT = NP_ // NS
BN = 2000


def _make_sc_agg(C, with_cnt):
    out_type = [jax.ShapeDtypeStruct((C * NP_, LANES), jnp.float32)]
    if with_cnt:
        out_type.append(jax.ShapeDtypeStruct((NC, NP_), jnp.float32))
    mesh = plsc.VectorSubcoreMesh(core_axis_name="c", subcore_axis_name="s",
                                  num_cores=NC, num_subcores=NS)
    scratch_types = [
        *[pltpu.VMEM((2, G, K), jnp.int32) for _ in range(2)],
        *[pltpu.VMEM((K, LANES), jnp.float32) for _ in range(2)],
        pltpu.VMEM((K,), jnp.float32),
        pltpu.VMEM_SHARED((NP_, LANES), jnp.float32),
        pltpu.VMEM_SHARED((NP_,), jnp.float32),
        *[pltpu.SemaphoreType.DMA for _ in range(6)],
    ]

    def body(xs, egrp, z2d, z1d, onesk, *rest):
        if with_cnt:
            agg_o, cnt_o = rest[0], rest[1]
            rest = rest[2:]
        else:
            agg_o = rest[0]
            rest = rest[1:]
        egb = rest[0:2]
        rows = rest[2:4]
        onev, agg_sp, cnt_sp = rest[4:7]
        isems = rest[7:9]
        gsems = rest[9:11]
        ssems = rest[11:13]
        ci = lax.axis_index("c")
        si = lax.axis_index("s")
        r0 = si * RPT

        if with_cnt:
            pltpu.sync_copy(onesk, onev)

        def cnt_on(i):
            return jnp.logical_xor(ci == 1, i < B // 2)

        for cc in range(C // NC):
            chunk = cc * NC + ci
            xs_c = xs.at[pl.ds(pl.multiple_of(chunk * N, 8), N)]

            def grp_issue(q, j):
                pltpu.async_copy(egrp.at[si, j], egb[q], isems[q])

            def grp_wait(q, j):
                pltpu.make_async_copy(egrp.at[si, j], egb[q], isems[q]).wait()

            def g_issue(q, p4, b):
                pltpu.async_copy(xs_c.at[egb[q].at[0, p4]], rows[b], gsems[b])

            def g_wait(q, p4, b):
                pltpu.make_async_copy(xs_c.at[egb[q].at[0, p4]], rows[b],
                                      gsems[b]).wait()

            def s_issue(q, p4, b, i):
                pltpu.async_copy(rows[b], agg_sp.at[egb[q].at[1, p4]],
                                 ssems[b], add=True)
                if with_cnt:
                    @pl.when(cnt_on(i))
                    def _():
                        pltpu.async_copy(onev, cnt_sp.at[egb[q].at[1, p4]],
                                         ssems[b], add=True)

            def s_wait(q, p4, b, i):
                pltpu.make_async_copy(rows[b], agg_sp.at[egb[q].at[1, p4]],
                                      ssems[b]).wait()
                if with_cnt:
                    @pl.when(cnt_on(i))
                    def _():
                        pltpu.make_async_copy(onev, cnt_sp.at[egb[q].at[1, p4]],
                                              ssems[b]).wait()

            pltpu.sync_copy(z2d.at[pl.ds(r0, RPT)], agg_sp.at[pl.ds(r0, RPT)])
            if with_cnt:
                @pl.when(si == 0)
                def _():
                    pltpu.sync_copy(z1d, cnt_sp)
            plsc.subcore_barrier()

            grp_issue(0, 0)

            def group_body(gidx, q, last):
                grp_wait(q, gidx)
                for p5 in range(G):
                    i = G * gidx + p5
                    b = (q + p5) % 2

                    @pl.when(i >= 2)
                    def _():
                        qq2 = q if p5 >= 2 else 1 - q
                        s_wait(qq2, (p5 - 2) % G, b, i - 2)
                    g_issue(q, p5, b)
                    qq, pp = (q, p5 - 1) if p5 else (1 - q, G - 1)

                    @pl.when(i >= 1)
                    def _():
                        g_wait(qq, pp, 1 - b)
                        s_issue(qq, pp, 1 - b, i - 1)

                if not last:
                    grp_issue(1 - q, gidx + 1)

            def step(g, _):
                for q in range(2):
                    group_body(2 * g + q, q, False)
                return 0

            lax.fori_loop(0, NGRP // 2, step, 0)
            group_body(NGRP - 1, 0, True)
            bL = (0 + G - 1) % 2
            g_wait(0, G - 1, bL)
            s_issue(0, G - 1, bL, B - 1)
            s_wait(0, G - 2, 1 - bL, B - 2)
            s_wait(0, G - 1, bL, B - 1)
            plsc.subcore_barrier()

            o0 = chunk * NP_ + r0
            pltpu.sync_copy(agg_sp.at[pl.ds(r0, RPT)], agg_o.at[pl.ds(o0, RPT)])
            if with_cnt:
                @pl.when(si == 0)
                def _():
                    pltpu.sync_copy(cnt_sp, cnt_o.at[ci])
            if cc + 1 < C // NC:
                plsc.subcore_barrier()

    return pl.kernel(body, out_type=out_type, mesh=mesh,
                     scratch_types=scratch_types)


_sc_agg2 = _make_sc_agg(2, with_cnt=True)
_sc_agg4 = _make_sc_agg(4, with_cnt=False)


def _tc0_body(agg_ref, cnt_ref, x_ref, wl_ref, wr_ref, b_ref, out_ref):
    inv = 1.0 / jnp.maximum(cnt_ref[0] + cnt_ref[1], 1.0)
    wl = wl_ref[...]
    acc = jnp.dot(x_ref[...], wr_ref[...], preferred_element_type=jnp.float32)
    for c in range(D_IN // LANES):
        mean_c = (agg_ref[c] * inv).astype(jnp.bfloat16)
        acc += jnp.dot(mean_c, wl[c * LANES:(c + 1) * LANES, :],
                       preferred_element_type=jnp.float32)
    h = jnp.maximum(acc + b_ref[...], 0.0)
    for c in range(HID // LANES):
        out_ref[c] = h[:, c * LANES:(c + 1) * LANES]


_tc_layer0 = pl.pallas_call(
    _tc0_body,
    grid=(N // BN,),
    in_specs=[
        pl.BlockSpec((D_IN // LANES, BN, LANES), lambda i: (0, i, 0)),
        pl.BlockSpec((NC, BN, 1), lambda i: (0, i, 0)),
        pl.BlockSpec((BN, D_IN), lambda i: (i, 0)),
        pl.BlockSpec((D_IN, HID), lambda i: (0, 0)),
        pl.BlockSpec((D_IN, HID), lambda i: (0, 0)),
        pl.BlockSpec((1, HID), lambda i: (0, 0)),
    ],
    out_specs=pl.BlockSpec((HID // LANES, BN, LANES), lambda i: (0, i, 0)),
    out_shape=jax.ShapeDtypeStruct((HID // LANES, N, LANES), jnp.float32),
)


def _tc1_body(agg_ref, cnt_ref, h_ref, wl_ref, wr_ref, b_ref, out_ref):
    inv = 1.0 / jnp.maximum(cnt_ref[0] + cnt_ref[1], 1.0)
    wl = wl_ref[...]
    wr = wr_ref[...]
    acc = jnp.broadcast_to(b_ref[...], (BN, HID))
    for c in range(HID // LANES):
        mean_c = (agg_ref[c] * inv).astype(jnp.bfloat16)
        acc += jnp.dot(mean_c, wl[c * LANES:(c + 1) * LANES, :],
                       preferred_element_type=jnp.float32)
        acc += jnp.dot(h_ref[c].astype(jnp.bfloat16),
                       wr[c * LANES:(c + 1) * LANES, :],
                       preferred_element_type=jnp.float32)
    out_ref[...] = jnp.maximum(acc, 0.0)


_tc_layer1 = pl.pallas_call(
    _tc1_body,
    grid=(N // BN,),
    in_specs=[
        pl.BlockSpec((HID // LANES, BN, LANES), lambda i: (0, i, 0)),
        pl.BlockSpec((NC, BN, 1), lambda i: (0, i, 0)),
        pl.BlockSpec((HID // LANES, BN, LANES), lambda i: (0, i, 0)),
        pl.BlockSpec((HID, HID), lambda i: (0, 0)),
        pl.BlockSpec((HID, HID), lambda i: (0, 0)),
        pl.BlockSpec((1, HID), lambda i: (0, 0)),
    ],
    out_specs=pl.BlockSpec((BN, HID), lambda i: (i, 0)),
    out_shape=jax.ShapeDtypeStruct((N, HID), jnp.float32),
)


def kernel(x, edge_index, W0_l, W0_r, b0, W1_l, W1_r, b1):
    xs0 = x.reshape(N, D_IN // LANES, LANES).transpose(1, 0, 2)
    xs0 = xs0.reshape((D_IN // LANES) * N, LANES)
    z2d = jnp.zeros((NP_, LANES), jnp.float32)
    z1d = jnp.zeros((NP_,), jnp.float32)
    onesk = jnp.ones((K,), jnp.float32)

    egrp = edge_index.reshape(2, NS, NGRP, G, K).transpose(1, 2, 0, 3, 4)
    agg0, cnt = _sc_agg2(xs0, egrp, z2d, z1d, onesk)
    cnt2 = cnt.reshape(NC, NP_, 1)
    bf = jnp.bfloat16
    h_split = _tc_layer0(agg0.reshape(D_IN // LANES, NP_, LANES), cnt2,
                         x.astype(bf), W0_l.astype(bf), W0_r.astype(bf),
                         b0.reshape(1, HID))
    agg1, = _sc_agg4(h_split.reshape((HID // LANES) * N, LANES), egrp,
                     z2d, z1d, onesk)
    out = _tc_layer1(agg1.reshape(HID // LANES, NP_, LANES), cnt2, h_split,
                     W1_l.astype(bf), W1_r.astype(bf), b1.reshape(1, HID))
    return out
